# Initial kernel scaffold; baseline (speedup 1.0000x reference)
import jax
import jax.numpy as jnp
from jax.experimental import pallas as pl


def kernel(feat_matrix, adj_matrix, get_item_index, set_index, val_index, mask_matrix, params):
    raise NotImplementedError("write your pallas kernel here")


# speedup vs baseline: 44.2953x; 44.2953x over previous
import functools

import jax
import jax.numpy as jnp
from jax.experimental import pallas as pl

F32 = jnp.float32


def _deg_kernel(a_ref, o_ref):
    r = pl.program_id(1)

    @pl.when(r == 0)
    def _init():
        o_ref[...] = jnp.zeros_like(o_ref)

    o_ref[...] += jnp.sum((a_ref[...] != 0.0).astype(F32), axis=0,
                          keepdims=True)


def _degrees(a2):
    n, m = a2.shape
    br = min(512, n)
    bc = min(2048, m)
    return pl.pallas_call(
        _deg_kernel,
        grid=(m // bc, n // br),
        in_specs=[pl.BlockSpec((br, bc), lambda c, r: (r, c))],
        out_specs=pl.BlockSpec((1, bc), lambda c, r: (0, c)),
        out_shape=jax.ShapeDtypeStruct((1, m), F32),
    )(a2)


def _mm_kernel(x_ref, w_ref, o_ref):
    o_ref[...] = jnp.dot(x_ref[...], w_ref[...], preferred_element_type=F32)


def _matmul(x, w):
    n, d = x.shape
    h = w.shape[1]
    bn = min(512, n)
    return pl.pallas_call(
        _mm_kernel,
        grid=(n // bn,),
        in_specs=[pl.BlockSpec((bn, d), lambda i: (i, 0)),
                  pl.BlockSpec((d, h), lambda i: (0, 0))],
        out_specs=pl.BlockSpec((bn, h), lambda i: (i, 0)),
        out_shape=jax.ShapeDtypeStruct((n, h), F32),
    )(x, w)


def _conv_kernel(adj_ref, xws_ref, dinvs_ref, xwi_ref, dinvi_ref, b_ref,
                 o_ref, *, ns):
    s = pl.program_id(1)

    @pl.when(s == 0)
    def _init():
        o_ref[...] = jnp.zeros_like(o_ref)

    m = (adj_ref[...] != 0.0).astype(F32)
    y = dinvs_ref[...] * xws_ref[...]
    o_ref[...] += jax.lax.dot_general(
        m, y, (((0,), (0,)), ((), ())), preferred_element_type=F32)

    @pl.when(s == ns - 1)
    def _fin():
        di = dinvi_ref[...]
        o_ref[...] = di * (o_ref[...] + di * xwi_ref[...]) + b_ref[...]


def _gcn_conv(adjk, xw, dinv_col, b2):
    n, h = xw.shape
    bs = min(512, n)
    bi = min(512, n)
    ns = n // bs
    return pl.pallas_call(
        functools.partial(_conv_kernel, ns=ns),
        grid=(n // bi, ns),
        in_specs=[
            pl.BlockSpec((bs, bi), lambda i, s: (s, i)),
            pl.BlockSpec((bs, h), lambda i, s: (s, 0)),
            pl.BlockSpec((bs, 1), lambda i, s: (s, 0)),
            pl.BlockSpec((bi, h), lambda i, s: (i, 0)),
            pl.BlockSpec((bi, 1), lambda i, s: (i, 0)),
            pl.BlockSpec((1, h), lambda i, s: (0, 0)),
        ],
        out_specs=pl.BlockSpec((bi, h), lambda i, s: (i, 0)),
        out_shape=jax.ShapeDtypeStruct((n, h), F32),
    )(adjk, xw, dinv_col, xw, dinv_col, b2)


def _readout_kernel(g0_ref, g1_ref, wih_f_ref, whh_f_ref, bf_ref,
                    wih_b_ref, whh_b_ref, bb_ref, wrof_ref, wrob_ref,
                    wo_ref, bo_ref, o_ref, *, h):
    g0 = g0_ref[...]
    g1 = g1_ref[...]

    def gates(pre):
        i = jax.nn.sigmoid(pre[:, 0 * h:1 * h])
        f = jax.nn.sigmoid(pre[:, 1 * h:2 * h])
        g = jnp.tanh(pre[:, 2 * h:3 * h])
        o = jax.nn.sigmoid(pre[:, 3 * h:4 * h])
        return i, f, g, o

    def step0(x, wih, b):
        i, f, g, o = gates(jnp.dot(x, wih, preferred_element_type=F32) + b)
        c = i * g
        return o * jnp.tanh(c), c

    def step(x, hp, cp, wih, whh, b):
        pre = (jnp.dot(x, wih, preferred_element_type=F32)
               + jnp.dot(hp, whh, preferred_element_type=F32) + b)
        i, f, g, o = gates(pre)
        c = f * cp + i * g
        return o * jnp.tanh(c), c

    wih_f = wih_f_ref[...]
    bf = bf_ref[...]
    wih_b = wih_b_ref[...]
    bb = bb_ref[...]
    f0, cf = step0(g0, wih_f, bf)
    f1, _ = step(g1, f0, cf, wih_f, whh_f_ref[...], bf)
    b1, cb = step0(g1, wih_b, bb)
    b0, _ = step(g0, b1, cb, wih_b, whh_b_ref[...], bb)

    wrof = wrof_ref[...]
    wrob = wrob_ref[...]
    s0 = (jnp.sum(f0 * wrof, axis=1, keepdims=True)
          + jnp.sum(b0 * wrob, axis=1, keepdims=True))
    s1 = (jnp.sum(f1 * wrof, axis=1, keepdims=True)
          + jnp.sum(b1 * wrob, axis=1, keepdims=True))
    mx = jnp.maximum(s0, s1)
    e0 = jnp.exp(s0 - mx)
    e1 = jnp.exp(s1 - mx)
    inv = 1.0 / (e0 + e1)
    pooled = (e0 * inv) * g0 + (e1 * inv) * g1

    logits = jnp.dot(pooled, wo_ref[...], preferred_element_type=F32) \
        + bo_ref[...]
    lm = jnp.max(logits, axis=1, keepdims=True)
    ex = jnp.exp(logits - lm)
    o_ref[...] = ex / jnp.sum(ex, axis=1, keepdims=True)


def _readout(g0, g1, params):
    n, h = g0.shape
    out_dim = params['Wo'].shape[0]
    bn = min(256, n)
    wih_f = params['Wih_f'].T
    whh_f = params['Whh_f'].T
    bf = (params['bih_f'] + params['bhh_f']).reshape(1, 4 * h)
    wih_b = params['Wih_b'].T
    whh_b = params['Whh_b'].T
    bb = (params['bih_b'] + params['bhh_b']).reshape(1, 4 * h)
    wrof = params['Wro'][:, :h]
    wrob = params['Wro'][:, h:]
    wo = params['Wo'].T
    bo = params['bo'].reshape(1, out_dim)
    full = lambda shape: pl.BlockSpec(shape, lambda i: (0, 0))
    return pl.pallas_call(
        functools.partial(_readout_kernel, h=h),
        grid=(n // bn,),
        in_specs=[
            pl.BlockSpec((bn, h), lambda i: (i, 0)),
            pl.BlockSpec((bn, h), lambda i: (i, 0)),
            full((h, 4 * h)), full((h, 4 * h)), full((1, 4 * h)),
            full((h, 4 * h)), full((h, 4 * h)), full((1, 4 * h)),
            full((1, h)), full((1, h)),
            full((h, out_dim)), full((1, out_dim)),
        ],
        out_specs=pl.BlockSpec((bn, out_dim), lambda i: (i, 0)),
        out_shape=jax.ShapeDtypeStruct((n, out_dim), F32),
    )(g0, g1, wih_f, whh_f, bf, wih_b, whh_b, bb, wrof, wrob, wo, bo)


def kernel(feat_matrix, adj_matrix, get_item_index, set_index, val_index,
           mask_matrix, params):
    n = feat_matrix.shape[0]
    k_graphs = adj_matrix.shape[2]

    a2 = adj_matrix.reshape(n, n * k_graphs)
    deg = _degrees(a2).reshape(n, k_graphs) + 1.0
    dinv = jax.lax.rsqrt(deg)

    gnn = []
    for k in range(k_graphs):
        adjk = adj_matrix[:, :, k]
        dk = dinv[:, k:k + 1]
        cur = feat_matrix
        for (w, b) in params['gcn'][k]:
            xw = _matmul(cur, w)
            cur = _gcn_conv(adjk, xw, dk, b.reshape(1, -1))
        gnn.append(cur)

    return _readout(gnn[0], gnn[1], params)


# speedup vs baseline: 63.2128x; 1.4271x over previous
import functools

import jax
import jax.numpy as jnp
from jax.experimental import pallas as pl

F32 = jnp.float32


def _pack_kernel(a_ref, p_ref, d_ref, *, nr):
    r = pl.program_id(1)

    @pl.when(r == 0)
    def _init():
        d_ref[...] = jnp.zeros_like(d_ref)

    m = (a_ref[...] != 0.0)
    d_ref[...] += jnp.sum(m.astype(F32), axis=0, keepdims=True)

    mi = m.astype(jnp.int32)
    br, bc = mi.shape
    m3 = mi.reshape(br // 32, 32, bc)
    sh = jax.lax.broadcasted_iota(jnp.int32, m3.shape, 1)
    p_ref[...] = jnp.sum(m3 << sh, axis=1)

    @pl.when(r == nr - 1)
    def _fin():
        d_ref[...] = jax.lax.rsqrt(d_ref[...] + 1.0)


def _pack(a2):
    n, m = a2.shape
    br = min(512, n)
    bc = min(2048, m)
    nr = n // br
    return pl.pallas_call(
        functools.partial(_pack_kernel, nr=nr),
        grid=(m // bc, nr),
        in_specs=[pl.BlockSpec((br, bc), lambda c, r: (r, c))],
        out_specs=[
            pl.BlockSpec((br // 32, bc), lambda c, r: (r, c)),
            pl.BlockSpec((1, bc), lambda c, r: (0, c)),
        ],
        out_shape=[
            jax.ShapeDtypeStruct((n // 32, m), jnp.int32),
            jax.ShapeDtypeStruct((1, m), F32),
        ],
    )(a2)


def _mm_kernel(x_ref, w_ref, o_ref):
    o_ref[...] = jnp.dot(x_ref[...], w_ref[...], preferred_element_type=F32)


def _matmul(x, w):
    n, d = x.shape
    h = w.shape[1]
    bn = min(512, n)
    return pl.pallas_call(
        _mm_kernel,
        grid=(n // bn,),
        in_specs=[pl.BlockSpec((bn, d), lambda i: (i, 0)),
                  pl.BlockSpec((d, h), lambda i: (0, 0))],
        out_specs=pl.BlockSpec((bn, h), lambda i: (i, 0)),
        out_shape=jax.ShapeDtypeStruct((n, h), F32),
    )(x, w)


def _conv_kernel(p_ref, xws_ref, dinvs_ref, xwi_ref, dinvi_ref, b_ref,
                 o_ref, *, ns):
    s = pl.program_id(1)

    @pl.when(s == 0)
    def _init():
        o_ref[...] = jnp.zeros_like(o_ref)

    p = p_ref[...]
    bw, bi = p.shape
    pb = jnp.broadcast_to(p[:, None, :], (bw, 32, bi))
    sh = jax.lax.broadcasted_iota(jnp.int32, pb.shape, 1)
    m = ((pb >> sh) & 1).astype(F32).reshape(bw * 32, bi)
    y = dinvs_ref[...] * xws_ref[...]
    o_ref[...] += jax.lax.dot_general(
        m, y, (((0,), (0,)), ((), ())), preferred_element_type=F32)

    @pl.when(s == ns - 1)
    def _fin():
        di = dinvi_ref[...]
        o_ref[...] = di * (o_ref[...] + di * xwi_ref[...]) + b_ref[...]


def _gcn_conv(pk, xw, dinv_col, b2):
    n, h = xw.shape
    bs = min(512, n)
    bi = min(512, n)
    ns = n // bs
    return pl.pallas_call(
        functools.partial(_conv_kernel, ns=ns),
        grid=(n // bi, ns),
        in_specs=[
            pl.BlockSpec((bs // 32, bi), lambda i, s: (s, i)),
            pl.BlockSpec((bs, h), lambda i, s: (s, 0)),
            pl.BlockSpec((bs, 1), lambda i, s: (s, 0)),
            pl.BlockSpec((bi, h), lambda i, s: (i, 0)),
            pl.BlockSpec((bi, 1), lambda i, s: (i, 0)),
            pl.BlockSpec((1, h), lambda i, s: (0, 0)),
        ],
        out_specs=pl.BlockSpec((bi, h), lambda i, s: (i, 0)),
        out_shape=jax.ShapeDtypeStruct((n, h), F32),
    )(pk, xw, dinv_col, xw, dinv_col, b2)


def _readout_kernel(g0_ref, g1_ref, wih_f_ref, whh_f_ref, bf_ref,
                    wih_b_ref, whh_b_ref, bb_ref, wrof_ref, wrob_ref,
                    wo_ref, bo_ref, o_ref, *, h):
    g0 = g0_ref[...]
    g1 = g1_ref[...]

    def gates(pre):
        i = jax.nn.sigmoid(pre[:, 0 * h:1 * h])
        f = jax.nn.sigmoid(pre[:, 1 * h:2 * h])
        g = jnp.tanh(pre[:, 2 * h:3 * h])
        o = jax.nn.sigmoid(pre[:, 3 * h:4 * h])
        return i, f, g, o

    def step0(x, wih, b):
        i, f, g, o = gates(jnp.dot(x, wih, preferred_element_type=F32) + b)
        c = i * g
        return o * jnp.tanh(c), c

    def step(x, hp, cp, wih, whh, b):
        pre = (jnp.dot(x, wih, preferred_element_type=F32)
               + jnp.dot(hp, whh, preferred_element_type=F32) + b)
        i, f, g, o = gates(pre)
        c = f * cp + i * g
        return o * jnp.tanh(c), c

    wih_f = wih_f_ref[...]
    bf = bf_ref[...]
    wih_b = wih_b_ref[...]
    bb = bb_ref[...]
    f0, cf = step0(g0, wih_f, bf)
    f1, _ = step(g1, f0, cf, wih_f, whh_f_ref[...], bf)
    b1, cb = step0(g1, wih_b, bb)
    b0, _ = step(g0, b1, cb, wih_b, whh_b_ref[...], bb)

    wrof = wrof_ref[...]
    wrob = wrob_ref[...]
    s0 = (jnp.sum(f0 * wrof, axis=1, keepdims=True)
          + jnp.sum(b0 * wrob, axis=1, keepdims=True))
    s1 = (jnp.sum(f1 * wrof, axis=1, keepdims=True)
          + jnp.sum(b1 * wrob, axis=1, keepdims=True))
    mx = jnp.maximum(s0, s1)
    e0 = jnp.exp(s0 - mx)
    e1 = jnp.exp(s1 - mx)
    inv = 1.0 / (e0 + e1)
    pooled = (e0 * inv) * g0 + (e1 * inv) * g1

    logits = jnp.dot(pooled, wo_ref[...], preferred_element_type=F32) \
        + bo_ref[...]
    lm = jnp.max(logits, axis=1, keepdims=True)
    ex = jnp.exp(logits - lm)
    o_ref[...] = ex / jnp.sum(ex, axis=1, keepdims=True)


def _readout(g0, g1, params):
    n, h = g0.shape
    out_dim = params['Wo'].shape[0]
    bn = min(256, n)
    wih_f = params['Wih_f'].T
    whh_f = params['Whh_f'].T
    bf = (params['bih_f'] + params['bhh_f']).reshape(1, 4 * h)
    wih_b = params['Wih_b'].T
    whh_b = params['Whh_b'].T
    bb = (params['bih_b'] + params['bhh_b']).reshape(1, 4 * h)
    wrof = params['Wro'][:, :h]
    wrob = params['Wro'][:, h:]
    wo = params['Wo'].T
    bo = params['bo'].reshape(1, out_dim)
    full = lambda shape: pl.BlockSpec(shape, lambda i: (0, 0))
    return pl.pallas_call(
        functools.partial(_readout_kernel, h=h),
        grid=(n // bn,),
        in_specs=[
            pl.BlockSpec((bn, h), lambda i: (i, 0)),
            pl.BlockSpec((bn, h), lambda i: (i, 0)),
            full((h, 4 * h)), full((h, 4 * h)), full((1, 4 * h)),
            full((h, 4 * h)), full((h, 4 * h)), full((1, 4 * h)),
            full((1, h)), full((1, h)),
            full((h, out_dim)), full((1, out_dim)),
        ],
        out_specs=pl.BlockSpec((bn, out_dim), lambda i: (i, 0)),
        out_shape=jax.ShapeDtypeStruct((n, out_dim), F32),
    )(g0, g1, wih_f, whh_f, bf, wih_b, whh_b, bb, wrof, wrob, wo, bo)


def kernel(feat_matrix, adj_matrix, get_item_index, set_index, val_index,
           mask_matrix, params):
    n = feat_matrix.shape[0]
    k_graphs = adj_matrix.shape[2]

    a2 = adj_matrix.reshape(n, n * k_graphs)
    p2, dinv2 = _pack(a2)
    dinv = dinv2.reshape(n, k_graphs)
    p3 = p2.reshape(n // 32, n, k_graphs)

    gnn = []
    for k in range(k_graphs):
        pk = p3[:, :, k]
        dk = dinv[:, k:k + 1]
        cur = feat_matrix
        for (w, b) in params['gcn'][k]:
            xw = _matmul(cur, w)
            cur = _gcn_conv(pk, xw, dk, b.reshape(1, -1))
        gnn.append(cur)

    return _readout(gnn[0], gnn[1], params)


# speedup vs baseline: 63.2494x; 1.0006x over previous
import functools

import jax
import jax.numpy as jnp
from jax import lax
from jax.experimental import pallas as pl
from jax.experimental.pallas import tpu as pltpu
from jax.experimental.pallas import tpu_sc as plsc

F32 = jnp.float32

_SC_CORES = 2
_SC_SUBCORES = 16
_SC_LANES = 16


def _deinterleave_sc(p2):
    w_rows, c_cols = p2.shape
    i_cols = c_cols // 2
    nw = _SC_CORES * _SC_SUBCORES
    rows_per = w_rows // nw
    nj = i_cols // _SC_LANES
    mesh = plsc.VectorSubcoreMesh(core_axis_name="c", subcore_axis_name="s",
                                  num_cores=_SC_CORES,
                                  num_subcores=_SC_SUBCORES)

    @functools.partial(
        pl.kernel, mesh=mesh,
        out_type=[jax.ShapeDtypeStruct((w_rows, i_cols), jnp.int32),
                  jax.ShapeDtypeStruct((w_rows, i_cols), jnp.int32)],
        scratch_types=[pltpu.VMEM((c_cols,), jnp.int32),
                       pltpu.VMEM((i_cols,), jnp.int32),
                       pltpu.VMEM((i_cols,), jnp.int32)],
    )
    def dk(p2_hbm, p0_hbm, p1_hbm, row_v, o0_v, o1_v):
        wid = lax.axis_index("s") * _SC_CORES + lax.axis_index("c")
        lanes = lax.iota(jnp.int32, _SC_LANES)

        def row_body(r, carry):
            row = wid * rows_per + r
            pltpu.sync_copy(p2_hbm.at[row], row_v)

            vx = plsc.load_gather(row_v, [2 * lanes])
            o0_v[pl.ds(0, _SC_LANES)] = vx

            def j_body(j, carry2):
                o0_v[pl.ds(j * _SC_LANES, _SC_LANES)] = vx
                o1_v[pl.ds(j * _SC_LANES, _SC_LANES)] = vx
                return carry2

            lax.fori_loop(0, nj, j_body, 0)
            pltpu.sync_copy(o0_v, p0_hbm.at[row])
            pltpu.sync_copy(o1_v, p1_hbm.at[row])
            return carry

        lax.fori_loop(0, rows_per, row_body, 0)

    return dk(p2)


def _pack_kernel(a_ref, p_ref, d_ref, *, nr):
    r = pl.program_id(1)

    @pl.when(r == 0)
    def _init():
        d_ref[...] = jnp.zeros_like(d_ref)

    m = (a_ref[...] != 0.0)
    d_ref[...] += jnp.sum(m.astype(F32), axis=0, keepdims=True)

    mi = m.astype(jnp.int32)
    br, bc = mi.shape
    m3 = mi.reshape(br // 32, 32, bc)
    sh = jax.lax.broadcasted_iota(jnp.int32, m3.shape, 1)
    p_ref[...] = jnp.sum(m3 << sh, axis=1)

    @pl.when(r == nr - 1)
    def _fin():
        d_ref[...] = jax.lax.rsqrt(d_ref[...] + 1.0)


def _pack(a2):
    n, m = a2.shape
    br = min(512, n)
    bc = min(2048, m)
    nr = n // br
    return pl.pallas_call(
        functools.partial(_pack_kernel, nr=nr),
        grid=(m // bc, nr),
        in_specs=[pl.BlockSpec((br, bc), lambda c, r: (r, c))],
        out_specs=[
            pl.BlockSpec((br // 32, bc), lambda c, r: (r, c)),
            pl.BlockSpec((1, bc), lambda c, r: (0, c)),
        ],
        out_shape=[
            jax.ShapeDtypeStruct((n // 32, m), jnp.int32),
            jax.ShapeDtypeStruct((1, m), F32),
        ],
    )(a2)


def _mm_kernel(x_ref, w_ref, o_ref):
    o_ref[...] = jnp.dot(x_ref[...], w_ref[...], preferred_element_type=F32)


def _matmul(x, w):
    n, d = x.shape
    h = w.shape[1]
    bn = min(512, n)
    return pl.pallas_call(
        _mm_kernel,
        grid=(n // bn,),
        in_specs=[pl.BlockSpec((bn, d), lambda i: (i, 0)),
                  pl.BlockSpec((d, h), lambda i: (0, 0))],
        out_specs=pl.BlockSpec((bn, h), lambda i: (i, 0)),
        out_shape=jax.ShapeDtypeStruct((n, h), F32),
    )(x, w)


def _conv_kernel(p_ref, xws_ref, dinvs_ref, xwi_ref, dinvi_ref, b_ref,
                 o_ref, *, ns):
    s = pl.program_id(1)

    @pl.when(s == 0)
    def _init():
        o_ref[...] = jnp.zeros_like(o_ref)

    p = p_ref[...]
    bw, bi = p.shape
    pb = jnp.broadcast_to(p[:, None, :], (bw, 32, bi))
    sh = jax.lax.broadcasted_iota(jnp.int32, pb.shape, 1)
    m = ((pb >> sh) & 1).astype(F32).reshape(bw * 32, bi)
    y = dinvs_ref[...] * xws_ref[...]
    o_ref[...] += jax.lax.dot_general(
        m, y, (((0,), (0,)), ((), ())), preferred_element_type=F32)

    @pl.when(s == ns - 1)
    def _fin():
        di = dinvi_ref[...]
        o_ref[...] = di * (o_ref[...] + di * xwi_ref[...]) + b_ref[...]


def _gcn_conv(pk, xw, dinv_col, b2):
    n, h = xw.shape
    bs = min(512, n)
    bi = min(512, n)
    ns = n // bs
    return pl.pallas_call(
        functools.partial(_conv_kernel, ns=ns),
        grid=(n // bi, ns),
        in_specs=[
            pl.BlockSpec((bs // 32, bi), lambda i, s: (s, i)),
            pl.BlockSpec((bs, h), lambda i, s: (s, 0)),
            pl.BlockSpec((bs, 1), lambda i, s: (s, 0)),
            pl.BlockSpec((bi, h), lambda i, s: (i, 0)),
            pl.BlockSpec((bi, 1), lambda i, s: (i, 0)),
            pl.BlockSpec((1, h), lambda i, s: (0, 0)),
        ],
        out_specs=pl.BlockSpec((bi, h), lambda i, s: (i, 0)),
        out_shape=jax.ShapeDtypeStruct((n, h), F32),
    )(pk, xw, dinv_col, xw, dinv_col, b2)


def _readout_kernel(g0_ref, g1_ref, wih_f_ref, whh_f_ref, bf_ref,
                    wih_b_ref, whh_b_ref, bb_ref, wrof_ref, wrob_ref,
                    wo_ref, bo_ref, o_ref, *, h):
    g0 = g0_ref[...]
    g1 = g1_ref[...]

    def gates(pre):
        i = jax.nn.sigmoid(pre[:, 0 * h:1 * h])
        f = jax.nn.sigmoid(pre[:, 1 * h:2 * h])
        g = jnp.tanh(pre[:, 2 * h:3 * h])
        o = jax.nn.sigmoid(pre[:, 3 * h:4 * h])
        return i, f, g, o

    def step0(x, wih, b):
        i, f, g, o = gates(jnp.dot(x, wih, preferred_element_type=F32) + b)
        c = i * g
        return o * jnp.tanh(c), c

    def step(x, hp, cp, wih, whh, b):
        pre = (jnp.dot(x, wih, preferred_element_type=F32)
               + jnp.dot(hp, whh, preferred_element_type=F32) + b)
        i, f, g, o = gates(pre)
        c = f * cp + i * g
        return o * jnp.tanh(c), c

    wih_f = wih_f_ref[...]
    bf = bf_ref[...]
    wih_b = wih_b_ref[...]
    bb = bb_ref[...]
    f0, cf = step0(g0, wih_f, bf)
    f1, _ = step(g1, f0, cf, wih_f, whh_f_ref[...], bf)
    b1, cb = step0(g1, wih_b, bb)
    b0, _ = step(g0, b1, cb, wih_b, whh_b_ref[...], bb)

    wrof = wrof_ref[...]
    wrob = wrob_ref[...]
    s0 = (jnp.sum(f0 * wrof, axis=1, keepdims=True)
          + jnp.sum(b0 * wrob, axis=1, keepdims=True))
    s1 = (jnp.sum(f1 * wrof, axis=1, keepdims=True)
          + jnp.sum(b1 * wrob, axis=1, keepdims=True))
    mx = jnp.maximum(s0, s1)
    e0 = jnp.exp(s0 - mx)
    e1 = jnp.exp(s1 - mx)
    inv = 1.0 / (e0 + e1)
    pooled = (e0 * inv) * g0 + (e1 * inv) * g1

    logits = jnp.dot(pooled, wo_ref[...], preferred_element_type=F32) \
        + bo_ref[...]
    lm = jnp.max(logits, axis=1, keepdims=True)
    ex = jnp.exp(logits - lm)
    o_ref[...] = ex / jnp.sum(ex, axis=1, keepdims=True)


def _readout(g0, g1, params):
    n, h = g0.shape
    out_dim = params['Wo'].shape[0]
    bn = min(256, n)
    wih_f = params['Wih_f'].T
    whh_f = params['Whh_f'].T
    bf = (params['bih_f'] + params['bhh_f']).reshape(1, 4 * h)
    wih_b = params['Wih_b'].T
    whh_b = params['Whh_b'].T
    bb = (params['bih_b'] + params['bhh_b']).reshape(1, 4 * h)
    wrof = params['Wro'][:, :h]
    wrob = params['Wro'][:, h:]
    wo = params['Wo'].T
    bo = params['bo'].reshape(1, out_dim)
    full = lambda shape: pl.BlockSpec(shape, lambda i: (0, 0))
    return pl.pallas_call(
        functools.partial(_readout_kernel, h=h),
        grid=(n // bn,),
        in_specs=[
            pl.BlockSpec((bn, h), lambda i: (i, 0)),
            pl.BlockSpec((bn, h), lambda i: (i, 0)),
            full((h, 4 * h)), full((h, 4 * h)), full((1, 4 * h)),
            full((h, 4 * h)), full((h, 4 * h)), full((1, 4 * h)),
            full((1, h)), full((1, h)),
            full((h, out_dim)), full((1, out_dim)),
        ],
        out_specs=pl.BlockSpec((bn, out_dim), lambda i: (i, 0)),
        out_shape=jax.ShapeDtypeStruct((n, out_dim), F32),
    )(g0, g1, wih_f, whh_f, bf, wih_b, whh_b, bb, wrof, wrob, wo, bo)


def kernel(feat_matrix, adj_matrix, get_item_index, set_index, val_index,
           mask_matrix, params):
    n = feat_matrix.shape[0]
    k_graphs = adj_matrix.shape[2]

    a2 = adj_matrix.reshape(n, n * k_graphs)
    p2, dinv2 = _pack(a2)
    dinv = dinv2.reshape(n, k_graphs)
    p3 = p2.reshape(n // 32, n, k_graphs)
    planes = [p3[:, :, k] for k in range(k_graphs)]

    gnn = []
    for k in range(k_graphs):
        pk = planes[k]
        dk = dinv[:, k:k + 1]
        cur = feat_matrix
        for (w, b) in params['gcn'][k]:
            xw = _matmul(cur, w)
            cur = _gcn_conv(pk, xw, dk, b.reshape(1, -1))
        gnn.append(cur)

    return _readout(gnn[0], gnn[1], params)
